# baseline (speedup 1.0000x reference)
import jax
import jax.numpy as jnp
from jax import lax
from jax.experimental import pallas as pl
from jax.experimental.pallas import tpu as pltpu
from jax.experimental.pallas import tpu_sc as plsc

_ALPHA = 0.1

_NC, _NS, _L = 2, 16, 16
_NW = _NC * _NS
_M, _N = 16384, 4096
_K = 8192
_ROWS_W = _K // _NW
_CR, _CC = 8, 2048
_COLS_PER_ROWBAND = _N // _CC
_CHUNKS = (_ROWS_W // _CR) * _COLS_PER_ROWBAND
_NB = 4
_P = 2

_BM = 512


def _chunk_slice(x_hbm, base_row, k):
    r0 = base_row + (k // _COLS_PER_ROWBAND) * _CR
    c0 = (k % _COLS_PER_ROWBAND) * _CC
    return x_hbm.at[pl.ds(r0, _CR), pl.ds(c0, _CC)]


def _sc_body(x_hbm, o_hbm, b0, b1, b2, b3, si0, si1, si2, si3,
             so0, so1, so2, so3):
    wid = lax.axis_index("s") * _NC + lax.axis_index("c")
    base_row = wid * _ROWS_W
    bufs = (b0, b1, b2, b3)
    in_s = (si0, si1, si2, si3)
    out_s = (so0, so1, so2, so3)

    for k in range(_P):
        pltpu.async_copy(_chunk_slice(x_hbm, base_row, k), bufs[k], in_s[k])

    @pl.loop(0, _CHUNKS, step=_NB)
    def _outer(g):
        for b in range(_NB):
            k = g + b
            bn = (b + _P) % _NB
            @pl.when(k >= _NB - _P)
            def _():
                pltpu.make_async_copy(
                    bufs[bn], _chunk_slice(o_hbm, base_row, k), out_s[bn]
                ).wait()

            @pl.when(k + _P < _CHUNKS)
            def _():
                pltpu.async_copy(
                    _chunk_slice(x_hbm, base_row, k + _P), bufs[bn], in_s[bn]
                )

            pltpu.make_async_copy(
                _chunk_slice(x_hbm, base_row, k), bufs[b], in_s[b]
            ).wait()

            buf = bufs[b]

            @plsc.parallel_loop(0, _CC, step=_L, unroll=8)
            def _compute(i):
                for r in range(_CR):
                    v = buf[r, pl.ds(i, _L)]
                    buf[r, pl.ds(i, _L)] = v + jnp.where(v > -_ALPHA, -_ALPHA, _ALPHA)

            pltpu.async_copy(bufs[b], _chunk_slice(o_hbm, base_row, k), out_s[b])

    for k in range(_CHUNKS - (_NB - _P), _CHUNKS):
        b = k % _NB
        pltpu.make_async_copy(
            bufs[b], _chunk_slice(o_hbm, base_row, k), out_s[b]
        ).wait()


def _sc_part(X):
    mesh = plsc.VectorSubcoreMesh(core_axis_name="c", subcore_axis_name="s")
    return pl.kernel(
        _sc_body,
        mesh=mesh,
        out_type=jax.ShapeDtypeStruct((_K, _N), jnp.float32),
        scratch_types=(
            [pltpu.VMEM((_CR, _CC), jnp.float32)] * _NB
            + [pltpu.SemaphoreType.DMA] * (2 * _NB)
        ),
        compiler_params=pltpu.CompilerParams(use_tc_tiling_on_sc=True),
    )(X)


def _tc_blk_body(x_ref, o_ref):
    x = x_ref[...]
    o_ref[...] = x + jnp.where(x > -_ALPHA, -_ALPHA, _ALPHA)


def _tc_part(X):
    nblk = (_M - _K) // _BM
    off = _K // _BM
    return pl.pallas_call(
        _tc_blk_body,
        grid=(nblk,),
        in_specs=[pl.BlockSpec((_BM, _N), lambda i: (i + off, 0))],
        out_specs=pl.BlockSpec((_BM, _N), lambda i: (i, 0)),
        out_shape=jax.ShapeDtypeStruct((_M - _K, _N), jnp.float32),
    )(X)


def kernel(X):
    return jnp.concatenate([_sc_part(X), _tc_part(X)], axis=0)


# speedup vs baseline: 1.7376x; 1.7376x over previous
import jax
import jax.numpy as jnp
from jax import lax
from jax.experimental import pallas as pl
from jax.experimental.pallas import tpu as pltpu
from jax.experimental.pallas import tpu_sc as plsc

_ALPHA = 0.1

_NC, _NS, _L = 2, 16, 16
_NW = _NC * _NS
_M, _N = 16384, 4096
_K = _M
_ROWS_W = _K // _NW
_CR, _CC = 8, 1024
_COLS_PER_ROWBAND = _N // _CC
_CHUNKS = (_ROWS_W // _CR) * _COLS_PER_ROWBAND
_NB = 8
_P = 4

_BM = 512


def _chunk_slice(x_hbm, base_row, k):
    r0 = base_row + (k // _COLS_PER_ROWBAND) * _CR
    c0 = (k % _COLS_PER_ROWBAND) * _CC
    return x_hbm.at[pl.ds(r0, _CR), pl.ds(c0, _CC)]


def _sc_body(x_hbm, o_hbm, *scratch):
    wid = lax.axis_index("s") * _NC + lax.axis_index("c")
    base_row = wid * _ROWS_W
    bufs = scratch[:_NB]
    in_s = scratch[_NB:2 * _NB]
    out_s = scratch[2 * _NB:]

    for k in range(_P):
        pltpu.async_copy(_chunk_slice(x_hbm, base_row, k), bufs[k], in_s[k])

    @pl.loop(0, _CHUNKS, step=_NB)
    def _outer(g):
        for b in range(_NB):
            k = g + b
            bn = (b + _P) % _NB
            @pl.when(k >= _NB - _P)
            def _():
                pltpu.make_async_copy(
                    bufs[bn], _chunk_slice(o_hbm, base_row, k), out_s[bn]
                ).wait()

            @pl.when(k + _P < _CHUNKS)
            def _():
                pltpu.async_copy(
                    _chunk_slice(x_hbm, base_row, k + _P), bufs[bn], in_s[bn]
                )

            pltpu.make_async_copy(
                _chunk_slice(x_hbm, base_row, k), bufs[b], in_s[b]
            ).wait()

            buf = bufs[b]

            @plsc.parallel_loop(0, _CC, step=_L, unroll=8)
            def _compute(i):
                for r in range(_CR):
                    v = buf[r, pl.ds(i, _L)]
                    buf[r, pl.ds(i, _L)] = v + jnp.where(v > -_ALPHA, -_ALPHA, _ALPHA)

            pltpu.async_copy(bufs[b], _chunk_slice(o_hbm, base_row, k), out_s[b])

    for k in range(_CHUNKS - (_NB - _P), _CHUNKS):
        b = k % _NB
        pltpu.make_async_copy(
            bufs[b], _chunk_slice(o_hbm, base_row, k), out_s[b]
        ).wait()


def _sc_part(X):
    mesh = plsc.VectorSubcoreMesh(core_axis_name="c", subcore_axis_name="s")
    return pl.kernel(
        _sc_body,
        mesh=mesh,
        out_type=jax.ShapeDtypeStruct((_K, _N), jnp.float32),
        scratch_types=(
            [pltpu.VMEM((_CR, _CC), jnp.float32)] * _NB
            + [pltpu.SemaphoreType.DMA] * (2 * _NB)
        ),
        compiler_params=pltpu.CompilerParams(use_tc_tiling_on_sc=True),
    )(X)


def _tc_blk_body(x_ref, o_ref):
    x = x_ref[...]
    o_ref[...] = x + jnp.where(x > -_ALPHA, -_ALPHA, _ALPHA)


def _tc_part(X):
    nblk = (_M - _K) // _BM
    off = _K // _BM
    return pl.pallas_call(
        _tc_blk_body,
        grid=(nblk,),
        in_specs=[pl.BlockSpec((_BM, _N), lambda i: (i + off, 0))],
        out_specs=pl.BlockSpec((_BM, _N), lambda i: (i, 0)),
        out_shape=jax.ShapeDtypeStruct((_M - _K, _N), jnp.float32),
    )(X)


def kernel(X):
    return _sc_part(X)


# speedup vs baseline: 1.7424x; 1.0028x over previous
import jax
import jax.numpy as jnp
from jax import lax
from jax.experimental import pallas as pl
from jax.experimental.pallas import tpu as pltpu
from jax.experimental.pallas import tpu_sc as plsc

_ALPHA = 0.1

_NC, _NS, _L = 2, 16, 16
_NW = _NC * _NS
_M, _N = 16384, 4096
_ROWS_W = _M // _NW
_CR, _CC = 8, 2048
_COLS_PER_ROWBAND = _N // _CC
_CHUNKS = (_ROWS_W // _CR) * _COLS_PER_ROWBAND
_NB = 4
_P = 2


def _chunk_slice(x_hbm, base_row, k):
    r0 = base_row + (k // _COLS_PER_ROWBAND) * _CR
    c0 = (k % _COLS_PER_ROWBAND) * _CC
    return x_hbm.at[pl.ds(r0, _CR), pl.ds(c0, _CC)]


def _sc_body(x_hbm, o_hbm, *scratch):
    wid = lax.axis_index("s") * _NC + lax.axis_index("c")
    base_row = wid * _ROWS_W
    bufs = scratch[:_NB]
    in_s = scratch[_NB:2 * _NB]
    out_s = scratch[2 * _NB:]

    for k in range(_P):
        pltpu.async_copy(_chunk_slice(x_hbm, base_row, k), bufs[k], in_s[k])

    @pl.loop(0, _CHUNKS, step=_NB)
    def _outer(g):
        for b in range(_NB):
            k = g + b
            bn = (b + _P) % _NB
            @pl.when(k >= _NB - _P)
            def _():
                pltpu.make_async_copy(
                    bufs[bn], _chunk_slice(o_hbm, base_row, k), out_s[bn]
                ).wait()

            @pl.when(k + _P < _CHUNKS)
            def _():
                pltpu.async_copy(
                    _chunk_slice(x_hbm, base_row, k + _P), bufs[bn], in_s[bn]
                )

            pltpu.make_async_copy(
                _chunk_slice(x_hbm, base_row, k), bufs[b], in_s[b]
            ).wait()

            buf = bufs[b]

            @plsc.parallel_loop(0, _CC, step=_L, unroll=8)
            def _compute(i):
                for r in range(_CR):
                    v = buf[r, pl.ds(i, _L)]
                    buf[r, pl.ds(i, _L)] = v + jnp.where(v > -_ALPHA, -_ALPHA, _ALPHA)

            pltpu.async_copy(bufs[b], _chunk_slice(o_hbm, base_row, k), out_s[b])

    for k in range(_CHUNKS - (_NB - _P), _CHUNKS):
        b = k % _NB
        pltpu.make_async_copy(
            bufs[b], _chunk_slice(o_hbm, base_row, k), out_s[b]
        ).wait()


def kernel(X):
    mesh = plsc.VectorSubcoreMesh(core_axis_name="c", subcore_axis_name="s")
    return pl.kernel(
        _sc_body,
        mesh=mesh,
        out_type=jax.ShapeDtypeStruct((_M, _N), jnp.float32),
        scratch_types=(
            [pltpu.VMEM((_CR, _CC), jnp.float32)] * _NB
            + [pltpu.SemaphoreType.DMA] * (2 * _NB)
        ),
        compiler_params=pltpu.CompilerParams(use_tc_tiling_on_sc=True),
    )(X)
